# Initial kernel scaffold; baseline (speedup 1.0000x reference)
import functools

import jax
import jax.numpy as jnp
from jax import lax
from jax.experimental import pallas as pl
from jax.experimental.pallas import tpu as pltpu
from jax.experimental.pallas import tpu_sc as plsc

B = 8
N = 2048
NZ = 128
DIM = 128
K = 10
HALF = DIM // 2
GF = 512
EPS = 1e-5
NT = 512
RT = 256
CH = 128
R = B * K * N

F32 = jnp.float32


def _lrelu(v, s):
    return jnp.where(v >= 0, v, s * v)


def _style_body(xr_ref, zr_ref, w1x_ref, w1z_ref, b1_ref, w2_ref, b2_ref,
                wa_ref, wc1_ref, wc2_ref,
                style_ref, a_ref, c1_ref, c2_ref):
    xr = xr_ref[0]
    zr = zr_ref[0]
    nrm = jnp.sqrt(jnp.sum(zr * zr, axis=1, keepdims=True))
    zn = zr / (nrm + 1e-8)
    s = (jnp.dot(xr, w1x_ref[...], preferred_element_type=F32)
         + jnp.dot(zn, w1z_ref[...], preferred_element_type=F32) + b1_ref[...])
    s = _lrelu(s, 0.01)
    s = jnp.dot(s, w2_ref[...], preferred_element_type=F32) + b2_ref[...]
    s = _lrelu(s, 0.01)
    style_ref[0] = s
    a_ref[0] = jnp.dot(xr, wa_ref[...], preferred_element_type=F32)
    c1_ref[0] = jnp.dot(xr, wc1_ref[...], preferred_element_type=F32)
    c2_ref[0] = jnp.dot(xr, wc2_ref[...], preferred_element_type=F32)


def _style_tables(x, z, w1x, w1z, b1, w2, b2, wa, wc1, wc2):
    nt = N // NT
    return pl.pallas_call(
        _style_body,
        grid=(B, nt),
        in_specs=[
            pl.BlockSpec((1, NT, 3), lambda b, i: (b, i, 0)),
            pl.BlockSpec((1, NT, NZ), lambda b, i: (b, i, 0)),
            pl.BlockSpec(w1x.shape, lambda b, i: (0, 0)),
            pl.BlockSpec(w1z.shape, lambda b, i: (0, 0)),
            pl.BlockSpec(b1.shape, lambda b, i: (0, 0)),
            pl.BlockSpec(w2.shape, lambda b, i: (0, 0)),
            pl.BlockSpec(b2.shape, lambda b, i: (0, 0)),
            pl.BlockSpec(wa.shape, lambda b, i: (0, 0)),
            pl.BlockSpec(wc1.shape, lambda b, i: (0, 0)),
            pl.BlockSpec(wc2.shape, lambda b, i: (0, 0)),
        ],
        out_specs=[
            pl.BlockSpec((1, NT, DIM), lambda b, i: (b, i, 0)),
            pl.BlockSpec((1, NT, HALF), lambda b, i: (b, i, 0)),
            pl.BlockSpec((1, NT, DIM), lambda b, i: (b, i, 0)),
            pl.BlockSpec((1, NT, DIM), lambda b, i: (b, i, 0)),
        ],
        out_shape=[
            jax.ShapeDtypeStruct((B, N, DIM), F32),
            jax.ShapeDtypeStruct((B, N, HALF), F32),
            jax.ShapeDtypeStruct((B, N, DIM), F32),
            jax.ShapeDtypeStruct((B, N, DIM), F32),
        ],
    )(x, z, w1x, w1z, b1, w2, b2, wa, wc1, wc2)


def _knn_body(xr_ref, xf_ref, idx_ref):
    xr = xr_ref[0]
    xf = xf_ref[0]
    dn = (((1,), (1,)), ((), ()))
    p = lax.dot_general(xr, xf, dn, preferred_element_type=F32)
    ones = jnp.ones((1, xf.shape[1]), F32)
    sqc = lax.dot_general(ones, xf * xf, dn, preferred_element_type=F32)
    sqr = jnp.sum(xr * xr, axis=1, keepdims=True)
    dist = (-2.0 * p + sqr) + sqc
    iota = lax.broadcasted_iota(jnp.int32, dist.shape, 1)
    boff = pl.program_id(0) * N
    cols = []
    for j in range(K + 1):
        m = jnp.min(dist, axis=1, keepdims=True)
        am = jnp.min(jnp.where(dist == m, iota, N), axis=1, keepdims=True)
        if j > 0:
            cols.append(am + boff)
        dist = jnp.where(iota == am, jnp.float32(jnp.inf), dist)
    idx_ref[0] = jnp.concatenate(cols, axis=1)


def _knn(xrows, d):
    nt = N // RT
    return pl.pallas_call(
        _knn_body,
        grid=(B, nt),
        in_specs=[
            pl.BlockSpec((1, RT, d), lambda b, i: (b, i, 0)),
            pl.BlockSpec((1, N, d), lambda b, i: (b, 0, 0)),
        ],
        out_specs=pl.BlockSpec((1, RT, K), lambda b, i: (b, i, 0)),
        out_shape=jax.ShapeDtypeStruct((B, N, K), jnp.int32),
    )(xrows, xrows)


def _sc_gather(a_tab, c_tab, idx_flat):
    info = plsc.get_sparse_core_info()
    nc, ns = info.num_cores, info.num_subcores
    nw = nc * ns
    per_w = R // nw
    nch = per_w // CH
    mesh = plsc.VectorSubcoreMesh(core_axis_name="c", subcore_axis_name="s")

    @functools.partial(
        pl.kernel, mesh=mesh,
        out_type=[jax.ShapeDtypeStruct((R, HALF), F32),
                  jax.ShapeDtypeStruct((R, DIM), F32)],
        scratch_types=[pltpu.VMEM((CH,), jnp.int32),
                       pltpu.VMEM((CH, HALF), F32),
                       pltpu.VMEM((CH, DIM), F32),
                       pltpu.SemaphoreType.DMA,
                       pltpu.SemaphoreType.DMA])
    def gk(a_hbm, c_hbm, idx_hbm, oa_hbm, oc_hbm, idx_v, abuf, cbuf, sa, sc):
        wid = lax.axis_index("s") * nc + lax.axis_index("c")

        def body(i, carry):
            base = wid * per_w + i * CH
            pltpu.sync_copy(idx_hbm.at[pl.ds(base, CH)], idx_v)
            cpa = pltpu.async_copy(a_hbm.at[idx_v], abuf, sa)
            cpc = pltpu.async_copy(c_hbm.at[idx_v], cbuf, sc)
            cpa.wait()
            cpc.wait()
            pltpu.sync_copy(abuf, oa_hbm.at[pl.ds(base, CH)])
            pltpu.sync_copy(cbuf, oc_hbm.at[pl.ds(base, CH)])
            return carry

        lax.fori_loop(0, nch, body, 0)

    return gk(a_tab, c_tab, idx_flat)


def _e1_body(ga_ref, gc_ref, a_ref, c1_ref, bw_ref, bx_ref, sw_ref, sh_ref):
    first = (pl.program_id(0) == 0) & (pl.program_id(1) == 0)
    aw = a_ref[0]
    c1r = c1_ref[0] + bx_ref[...]
    ws = jnp.zeros((1, HALF), F32)
    wq = jnp.zeros((1, HALF), F32)
    hs = jnp.zeros((1, DIM), F32)
    hq = jnp.zeros((1, DIM), F32)
    for k in range(K):
        w = ga_ref[0, k] - aw + bw_ref[...]
        ws += jnp.sum(w, axis=0, keepdims=True)
        wq += jnp.sum(w * w, axis=0, keepdims=True)
        h = gc_ref[0, k] + c1r
        hs += jnp.sum(h, axis=0, keepdims=True)
        hq += jnp.sum(h * h, axis=0, keepdims=True)

    @pl.when(first)
    def _():
        sw_ref[...] = jnp.zeros_like(sw_ref)
        sh_ref[...] = jnp.zeros_like(sh_ref)

    sw_ref[...] += jnp.concatenate([ws, wq], axis=0)
    sh_ref[...] += jnp.concatenate([hs, hq], axis=0)


def _e1(ga, gc, a, c1, bw, bx):
    nt = N // NT
    return pl.pallas_call(
        _e1_body,
        grid=(B, nt),
        in_specs=[
            pl.BlockSpec((1, K, NT, HALF), lambda b, i: (b, 0, i, 0)),
            pl.BlockSpec((1, K, NT, DIM), lambda b, i: (b, 0, i, 0)),
            pl.BlockSpec((1, NT, HALF), lambda b, i: (b, i, 0)),
            pl.BlockSpec((1, NT, DIM), lambda b, i: (b, i, 0)),
            pl.BlockSpec((1, HALF), lambda b, i: (0, 0)),
            pl.BlockSpec((1, DIM), lambda b, i: (0, 0)),
        ],
        out_specs=[
            pl.BlockSpec((2, HALF), lambda b, i: (0, 0)),
            pl.BlockSpec((2, DIM), lambda b, i: (0, 0)),
        ],
        out_shape=[
            jax.ShapeDtypeStruct((2, HALF), F32),
            jax.ShapeDtypeStruct((2, DIM), F32),
        ],
    )(ga, gc, a, c1, bw, bx)


def _e2_body(ga_ref, a_ref, scw_ref, shw_ref, w2w_ref, w2b_ref,
             w2_ref, s2_ref):
    first = (pl.program_id(0) == 0) & (pl.program_id(1) == 0)
    aw = a_ref[0]
    sc = scw_ref[...]
    sh = shw_ref[...]
    ss = jnp.zeros((1, DIM), F32)
    sq = jnp.zeros((1, DIM), F32)
    for k in range(K):
        u = _lrelu((ga_ref[0, k] - aw) * sc + sh, 0.2)
        w2k = jnp.dot(u, w2w_ref[...], preferred_element_type=F32) + w2b_ref[...]
        w2_ref[0, k] = w2k
        ss += jnp.sum(w2k, axis=0, keepdims=True)
        sq += jnp.sum(w2k * w2k, axis=0, keepdims=True)

    @pl.when(first)
    def _():
        s2_ref[...] = jnp.zeros_like(s2_ref)

    s2_ref[...] += jnp.concatenate([ss, sq], axis=0)


def _e2(ga, a, scw, shw, w2w, w2b):
    nt = N // NT
    return pl.pallas_call(
        _e2_body,
        grid=(B, nt),
        in_specs=[
            pl.BlockSpec((1, K, NT, HALF), lambda b, i: (b, 0, i, 0)),
            pl.BlockSpec((1, NT, HALF), lambda b, i: (b, i, 0)),
            pl.BlockSpec((1, HALF), lambda b, i: (0, 0)),
            pl.BlockSpec((1, HALF), lambda b, i: (0, 0)),
            pl.BlockSpec((HALF, DIM), lambda b, i: (0, 0)),
            pl.BlockSpec((1, DIM), lambda b, i: (0, 0)),
        ],
        out_specs=[
            pl.BlockSpec((1, K, NT, DIM), lambda b, i: (b, 0, i, 0)),
            pl.BlockSpec((2, DIM), lambda b, i: (0, 0)),
        ],
        out_shape=[
            jax.ShapeDtypeStruct((B, K, N, DIM), F32),
            jax.ShapeDtypeStruct((2, DIM), F32),
        ],
    )(ga, a, scw, shw, w2w, w2b)


def _e3_body(w2_ref, gc_ref, c1_ref, sc2_ref, sh2_ref, sch_ref, shh_ref,
             cot_ref, cob_ref, h_ref, sb_ref):
    first = pl.program_id(1) == 0
    sc2 = sc2_ref[...]
    sh2 = sh2_ref[...]
    acts = [_lrelu(w2_ref[0, k] * sc2 + sh2, 0.2) for k in range(K)]
    mx = acts[0]
    for k in range(1, K):
        mx = jnp.maximum(mx, acts[k])
    es = [jnp.exp(a - mx) for a in acts]
    tot = es[0]
    for k in range(1, K):
        tot = tot + es[k]
    c1r = c1_ref[0]
    sch = sch_ref[...]
    shh = shh_ref[...]
    out = jnp.zeros(c1r.shape, F32) + cob_ref[...]
    for k in range(K):
        w = es[k] / tot
        h = _lrelu((gc_ref[0, k] + c1r) * sch + shh, 0.2)
        out += jnp.dot(h * w, cot_ref[k], preferred_element_type=F32)
    h_ref[0] = out

    @pl.when(first)
    def _():
        sb_ref[...] = jnp.zeros_like(sb_ref)

    sb_ref[0] += jnp.concatenate(
        [jnp.sum(out, axis=0, keepdims=True),
         jnp.sum(out * out, axis=0, keepdims=True)], axis=0)


def _e3(w2, gc, c1, sc2, sh2, sch, shh, cot, cob):
    nt = N // NT
    return pl.pallas_call(
        _e3_body,
        grid=(B, nt),
        in_specs=[
            pl.BlockSpec((1, K, NT, DIM), lambda b, i: (b, 0, i, 0)),
            pl.BlockSpec((1, K, NT, DIM), lambda b, i: (b, 0, i, 0)),
            pl.BlockSpec((1, NT, DIM), lambda b, i: (b, i, 0)),
            pl.BlockSpec((1, DIM), lambda b, i: (0, 0)),
            pl.BlockSpec((1, DIM), lambda b, i: (0, 0)),
            pl.BlockSpec((1, DIM), lambda b, i: (0, 0)),
            pl.BlockSpec((1, DIM), lambda b, i: (0, 0)),
            pl.BlockSpec((K, DIM, DIM), lambda b, i: (0, 0, 0)),
            pl.BlockSpec((1, DIM), lambda b, i: (0, 0)),
        ],
        out_specs=[
            pl.BlockSpec((1, NT, DIM), lambda b, i: (b, i, 0)),
            pl.BlockSpec((1, 2, DIM), lambda b, i: (b, 0, 0)),
        ],
        out_shape=[
            jax.ShapeDtypeStruct((B, N, DIM), F32),
            jax.ShapeDtypeStruct((B, 2, DIM), F32),
        ],
    )(w2, gc, c1, sc2, sh2, sch, shh, cot, cob)


def _e4_block0_body(h_ref, st_ref, m_ref, r_ref, adw_ref, adb_ref,
                    wa_ref, wc1_ref, wc2_ref,
                    h1_ref, a_ref, c1_ref, c2_ref):
    xh = (h_ref[0] - m_ref[0]) * r_ref[0]
    s = jnp.dot(st_ref[0], adw_ref[...], preferred_element_type=F32) + adb_ref[...]
    o = _lrelu(s[:, :DIM] * xh + s[:, DIM:], 0.2)
    h1_ref[0] = o
    a_ref[0] = jnp.dot(o, wa_ref[...], preferred_element_type=F32)
    c1_ref[0] = jnp.dot(o, wc1_ref[...], preferred_element_type=F32)
    c2_ref[0] = jnp.dot(o, wc2_ref[...], preferred_element_type=F32)


def _e4_block0(h, st, m, r, adw, adb, wa, wc1, wc2):
    nt = N // NT
    return pl.pallas_call(
        _e4_block0_body,
        grid=(B, nt),
        in_specs=[
            pl.BlockSpec((1, NT, DIM), lambda b, i: (b, i, 0)),
            pl.BlockSpec((1, NT, DIM), lambda b, i: (b, i, 0)),
            pl.BlockSpec((1, 1, DIM), lambda b, i: (b, 0, 0)),
            pl.BlockSpec((1, 1, DIM), lambda b, i: (b, 0, 0)),
            pl.BlockSpec((DIM, 2 * DIM), lambda b, i: (0, 0)),
            pl.BlockSpec((1, 2 * DIM), lambda b, i: (0, 0)),
            pl.BlockSpec((DIM, HALF), lambda b, i: (0, 0)),
            pl.BlockSpec((DIM, DIM), lambda b, i: (0, 0)),
            pl.BlockSpec((DIM, DIM), lambda b, i: (0, 0)),
        ],
        out_specs=[
            pl.BlockSpec((1, NT, DIM), lambda b, i: (b, i, 0)),
            pl.BlockSpec((1, NT, HALF), lambda b, i: (b, i, 0)),
            pl.BlockSpec((1, NT, DIM), lambda b, i: (b, i, 0)),
            pl.BlockSpec((1, NT, DIM), lambda b, i: (b, i, 0)),
        ],
        out_shape=[
            jax.ShapeDtypeStruct((B, N, DIM), F32),
            jax.ShapeDtypeStruct((B, N, HALF), F32),
            jax.ShapeDtypeStruct((B, N, DIM), F32),
            jax.ShapeDtypeStruct((B, N, DIM), F32),
        ],
    )(h, st, m, r, adw, adb, wa, wc1, wc2)


def _e4_block1_body(h_ref, st_ref, m_ref, r_ref, adw_ref, adb_ref,
                    h2_ref, mx_ref):
    i = pl.program_id(1)
    xh = (h_ref[0] - m_ref[0]) * r_ref[0]
    s = jnp.dot(st_ref[0], adw_ref[...], preferred_element_type=F32) + adb_ref[...]
    o = _lrelu(s[:, :DIM] * xh + s[:, DIM:], 0.2)
    h2_ref[0] = o
    pm = jnp.max(o, axis=0, keepdims=True)

    @pl.when(i == 0)
    def _():
        mx_ref[0] = pm

    @pl.when(i > 0)
    def _():
        mx_ref[0] = jnp.maximum(mx_ref[0], pm)


def _e4_block1(h, st, m, r, adw, adb):
    nt = N // NT
    return pl.pallas_call(
        _e4_block1_body,
        grid=(B, nt),
        in_specs=[
            pl.BlockSpec((1, NT, DIM), lambda b, i: (b, i, 0)),
            pl.BlockSpec((1, NT, DIM), lambda b, i: (b, i, 0)),
            pl.BlockSpec((1, 1, DIM), lambda b, i: (b, 0, 0)),
            pl.BlockSpec((1, 1, DIM), lambda b, i: (b, 0, 0)),
            pl.BlockSpec((DIM, 2 * DIM), lambda b, i: (0, 0)),
            pl.BlockSpec((1, 2 * DIM), lambda b, i: (0, 0)),
        ],
        out_specs=[
            pl.BlockSpec((1, NT, DIM), lambda b, i: (b, i, 0)),
            pl.BlockSpec((1, 1, DIM), lambda b, i: (b, 0, 0)),
        ],
        out_shape=[
            jax.ShapeDtypeStruct((B, N, DIM), F32),
            jax.ShapeDtypeStruct((B, 1, DIM), F32),
        ],
    )(h, st, m, r, adw, adb)


def _g_body(mx_ref, w1_ref, b1_ref, g1_ref, be1_ref, w2_ref, b2_ref,
            g2_ref, be2_ref, fg_ref):
    f = jnp.dot(mx_ref[...], w1_ref[...], preferred_element_type=F32) + b1_ref[...]
    m = jnp.mean(f, axis=0, keepdims=True)
    v = jnp.mean((f - m) * (f - m), axis=0, keepdims=True)
    f = _lrelu((f - m) / jnp.sqrt(v + EPS) * g1_ref[...] + be1_ref[...], 0.01)
    f = jnp.dot(f, w2_ref[...], preferred_element_type=F32) + b2_ref[...]
    m = jnp.mean(f, axis=0, keepdims=True)
    v = jnp.mean((f - m) * (f - m), axis=0, keepdims=True)
    fg_ref[...] = _lrelu((f - m) / jnp.sqrt(v + EPS) * g2_ref[...] + be2_ref[...], 0.01)


def _g_head(mx, w1, b1, g1, be1, w2, b2, g2, be2):
    args = (mx, w1, b1, g1, be1, w2, b2, g2, be2)
    return pl.pallas_call(
        _g_body,
        grid=(1,),
        in_specs=[pl.BlockSpec(a.shape, lambda i: (0, 0)) for a in args],
        out_specs=pl.BlockSpec((B, GF), lambda i: (0, 0)),
        out_shape=jax.ShapeDtypeStruct((B, GF), F32),
    )(*args)


def _asm_body(fg_ref, h2_ref, o_ref):
    o_ref[0, :, :GF] = jnp.broadcast_to(fg_ref[...], (NT, GF))
    o_ref[0, :, GF:] = h2_ref[0]


def _assemble(fg, h2):
    nt = N // NT
    return pl.pallas_call(
        _asm_body,
        grid=(B, nt),
        in_specs=[
            pl.BlockSpec((1, GF), lambda b, i: (b, 0)),
            pl.BlockSpec((1, NT, DIM), lambda b, i: (b, i, 0)),
        ],
        out_specs=pl.BlockSpec((1, NT, GF + DIM), lambda b, i: (b, i, 0)),
        out_shape=jax.ShapeDtypeStruct((B, N, GF + DIM), F32),
    )(fg, h2)


def _bn_coeffs(sums, m_count, g, bshift):
    mean = sums[0] / m_count
    var = jnp.maximum(sums[1] / m_count - mean * mean, 0.0)
    scale = g / jnp.sqrt(var + EPS)
    shift = bshift - mean * scale
    return scale[None], shift[None]


def _edge_block(pfx, h_rows, d, a, c1, c2, style, p, adw, adb, extra):
    idx = _knn(h_rows, d)
    idxf = jnp.transpose(idx, (0, 2, 1)).reshape(R)
    ga, gc = _sc_gather(a.reshape(B * N, HALF), c2.reshape(B * N, DIM), idxf)
    ga = ga.reshape(B, K, N, HALF)
    gc = gc.reshape(B, K, N, DIM)
    bw = p[pfx + 'cw1_b'][None]
    bx = p[pfx + 'cx_b'][None]
    sw, shst = _e1(ga, gc, a, c1, bw, bx)
    m_cnt = float(B * K * N)
    scw, shw = _bn_coeffs(sw, m_cnt, p[pfx + 'bnw1_g'], p[pfx + 'bnw1_b'])
    shw = shw + p[pfx + 'cw1_b'][None] * scw
    w2, s2 = _e2(ga, a, scw, shw, p[pfx + 'cw2_w'].T, p[pfx + 'cw2_b'][None])
    sc2, sh2 = _bn_coeffs(s2, m_cnt, p[pfx + 'bnw2_g'], p[pfx + 'bnw2_b'])
    sch, shh = _bn_coeffs(shst, m_cnt, p[pfx + 'bnx_g'], p[pfx + 'bnx_b'])
    shh = shh + p[pfx + 'cx_b'][None] * sch
    cot = jnp.transpose(p[pfx + 'co_w'], (2, 1, 0))
    heb, sb = _e3(w2, gc, c1, sc2, sh2, sch, shh, cot, p[pfx + 'co_b'][None])
    mb = sb[:, 0, :] / N
    vb = jnp.maximum(sb[:, 1, :] / N - mb * mb, 0.0)
    rb = 1.0 / jnp.sqrt(vb + EPS)
    mb = mb[:, None, :]
    rb = rb[:, None, :]
    if extra:
        wa1 = p['eb1_cw1_w'].T
        cxw = p['eb1_cx_w']
        wc11 = (cxw[:, :DIM] - cxw[:, DIM:]).T
        wc21 = cxw[:, DIM:].T
        return _e4_block0(heb, style, mb, rb, adw, adb, wa1, wc11, wc21)
    return _e4_block1(heb, style, mb, rb, adw, adb)


def kernel(x, z, params):
    p = params
    w1 = p['head_w1']
    w1x = w1[:, :3].T
    w1z = w1[:, 3:].T
    cxw0 = p['eb0_cx_w']
    style, a0, c10, c20 = _style_tables(
        x, z, w1x, w1z, p['head_b1'][None], p['head_w2'].T, p['head_b2'][None],
        p['eb0_cw1_w'].T, (cxw0[:, :3] - cxw0[:, 3:]).T, cxw0[:, 3:].T)

    h1, a1, c11, c21 = _edge_block(
        'eb0_', x, 3, a0, c10, c20, style, p,
        p['ad0_w'].T, p['ad0_b'][None], extra=True)

    h2, mx = _edge_block(
        'eb1_', h1, DIM, a1, c11, c21, style, p,
        p['ad1_w'].T, p['ad1_b'][None], extra=False)

    fg = _g_head(mx.reshape(B, DIM),
                 p['g_w1'].T, p['g_b1'][None], p['g_bn1_g'][None],
                 p['g_bn1_b'][None],
                 p['g_w2'].T, p['g_b2'][None], p['g_bn2_g'][None],
                 p['g_bn2_b'][None])
    return _assemble(fg, h2)


# speedup vs baseline: 11.4873x; 11.4873x over previous
import functools

import jax
import jax.numpy as jnp
from jax import lax
from jax.experimental import pallas as pl
from jax.experimental.pallas import tpu as pltpu
from jax.experimental.pallas import tpu_sc as plsc

B = 8
N = 2048
NZ = 128
DIM = 128
K = 10
HALF = DIM // 2
GF = 512
EPS = 1e-5
NT = 512
RT = 256
CH = 128
R = B * K * N

F32 = jnp.float32


def _lrelu(v, s):
    return jnp.where(v >= 0, v, s * v)


def _style_body(xr_ref, zr_ref, w1x_ref, b1_ref, w2_ref, b2_ref,
                style_ref, tab_ref):
    xr = xr_ref[0]
    zr = zr_ref[0]
    nrm = jnp.sqrt(jnp.sum(zr * zr, axis=1, keepdims=True))
    zn = zr / (nrm + 1e-8)
    cat = jnp.concatenate([xr, zn], axis=1)
    s = jnp.dot(cat, w1x_ref[...], preferred_element_type=F32) + b1_ref[...]
    s = _lrelu(s, 0.01)
    s = jnp.dot(s, w2_ref[...], preferred_element_type=F32) + b2_ref[...]
    s = _lrelu(s, 0.01)
    style_ref[0] = s
    tab_ref[0] = jnp.concatenate(
        [xr, jnp.zeros((xr.shape[0], DIM - 3), F32)], axis=1)


def _style_tab0(x, z, w1x, b1, w2, b2):
    nt = N // NT
    return pl.pallas_call(
        _style_body,
        grid=(B, nt),
        in_specs=[
            pl.BlockSpec((1, NT, 3), lambda b, i: (b, i, 0)),
            pl.BlockSpec((1, NT, NZ), lambda b, i: (b, i, 0)),
            pl.BlockSpec(w1x.shape, lambda b, i: (0, 0)),
            pl.BlockSpec(b1.shape, lambda b, i: (0, 0)),
            pl.BlockSpec(w2.shape, lambda b, i: (0, 0)),
            pl.BlockSpec(b2.shape, lambda b, i: (0, 0)),
        ],
        out_specs=[
            pl.BlockSpec((1, NT, DIM), lambda b, i: (b, i, 0)),
            pl.BlockSpec((1, NT, DIM), lambda b, i: (b, i, 0)),
        ],
        out_shape=[
            jax.ShapeDtypeStruct((B, N, DIM), F32),
            jax.ShapeDtypeStruct((B, N, DIM), F32),
        ],
    )(x, z, w1x, b1, w2, b2)


def _topk_from_dist(dist, idx_ref):
    iota = lax.broadcasted_iota(jnp.int32, dist.shape, 1)
    boff = pl.program_id(0) * N
    cols = []
    for j in range(K + 1):
        m = jnp.min(dist, axis=1, keepdims=True)
        am = jnp.min(jnp.where(dist == m, iota, N), axis=1, keepdims=True)
        if j > 0:
            cols.append(am + boff)
        dist = jnp.where(iota == am, jnp.float32(jnp.inf), dist)
    idx_ref[0] = jnp.concatenate(cols, axis=1)


def _knn_body(xr_ref, xt_ref, sqr_ref, sqc_ref, idx_ref):
    xr = xr_ref[0]
    xt = xt_ref[0]
    p = jnp.dot(xr, xt, preferred_element_type=F32)
    dist = (-2.0 * p + sqr_ref[0]) + sqc_ref[0]
    _topk_from_dist(dist, idx_ref)


def _knn(xrows, xt, sq, d):
    nt = N // RT
    return pl.pallas_call(
        _knn_body,
        grid=(B, nt),
        in_specs=[
            pl.BlockSpec((1, RT, d), lambda b, i: (b, i, 0)),
            pl.BlockSpec((1, d, N), lambda b, i: (b, 0, 0)),
            pl.BlockSpec((1, RT, 1), lambda b, i: (b, i, 0)),
            pl.BlockSpec((1, 1, N), lambda b, i: (b, 0, 0)),
        ],
        out_specs=pl.BlockSpec((1, RT, K), lambda b, i: (b, i, 0)),
        out_shape=jax.ShapeDtypeStruct((B, N, K), jnp.int32),
    )(xrows, xt, sq[:, :, None], sq[:, None, :])


def _sc_gather(tab, idx_flat):
    info = plsc.get_sparse_core_info()
    nc, ns = info.num_cores, info.num_subcores
    nw = nc * ns
    per_w = R // nw
    nch = per_w // CH
    mesh = plsc.VectorSubcoreMesh(core_axis_name="c", subcore_axis_name="s")

    @functools.partial(
        pl.kernel, mesh=mesh,
        out_type=jax.ShapeDtypeStruct((R, DIM), F32),
        scratch_types=[pltpu.VMEM((CH,), jnp.int32),
                       pltpu.VMEM((CH, DIM), F32),
                       pltpu.SemaphoreType.DMA])
    def gk(t_hbm, idx_hbm, o_hbm, idx_v, rbuf, sem):
        wid = lax.axis_index("s") * nc + lax.axis_index("c")

        def body(i, carry):
            base = wid * per_w + i * CH
            pltpu.sync_copy(idx_hbm.at[pl.ds(base, CH)], idx_v)
            pltpu.async_copy(t_hbm.at[idx_v], rbuf, sem).wait()
            pltpu.sync_copy(rbuf, o_hbm.at[pl.ds(base, CH)])
            return carry

        lax.fori_loop(0, nch, body, 0)

    return gk(tab, idx_flat)


def _e1_body(g_ref, tab_ref, w1p_ref, cxt_ref, bw_ref, bx_ref,
             sw_ref, sh_ref):
    first = (pl.program_id(0) == 0) & (pl.program_id(1) == 0)
    d = cxt_ref.shape[0] // 2
    cen = tab_ref[0]
    cend = cen[:, :d]
    ws = jnp.zeros((1, HALF), F32)
    wq = jnp.zeros((1, HALF), F32)
    hs = jnp.zeros((1, DIM), F32)
    hq = jnp.zeros((1, DIM), F32)
    for k in range(K):
        rel = g_ref[0, k] - cen
        w = jnp.dot(rel, w1p_ref[...], preferred_element_type=F32) + bw_ref[...]
        ws += jnp.sum(w, axis=0, keepdims=True)
        wq += jnp.sum(w * w, axis=0, keepdims=True)
        ee = jnp.concatenate([cend, rel[:, :d]], axis=1)
        h = jnp.dot(ee, cxt_ref[...], preferred_element_type=F32) + bx_ref[...]
        hs += jnp.sum(h, axis=0, keepdims=True)
        hq += jnp.sum(h * h, axis=0, keepdims=True)

    @pl.when(first)
    def _():
        sw_ref[...] = jnp.zeros_like(sw_ref)
        sh_ref[...] = jnp.zeros_like(sh_ref)

    sw_ref[...] += jnp.concatenate([ws, wq], axis=0)
    sh_ref[...] += jnp.concatenate([hs, hq], axis=0)


def _e1(g, tab, w1p, cxt, bw, bx):
    nt = N // NT
    return pl.pallas_call(
        _e1_body,
        grid=(B, nt),
        in_specs=[
            pl.BlockSpec((1, K, NT, DIM), lambda b, i: (b, 0, i, 0)),
            pl.BlockSpec((1, NT, DIM), lambda b, i: (b, i, 0)),
            pl.BlockSpec((DIM, HALF), lambda b, i: (0, 0)),
            pl.BlockSpec(cxt.shape, lambda b, i: (0, 0)),
            pl.BlockSpec((1, HALF), lambda b, i: (0, 0)),
            pl.BlockSpec((1, DIM), lambda b, i: (0, 0)),
        ],
        out_specs=[
            pl.BlockSpec((2, HALF), lambda b, i: (0, 0)),
            pl.BlockSpec((2, DIM), lambda b, i: (0, 0)),
        ],
        out_shape=[
            jax.ShapeDtypeStruct((2, HALF), F32),
            jax.ShapeDtypeStruct((2, DIM), F32),
        ],
    )(g, tab, w1p, cxt, bw, bx)


def _e1v_body(g_ref, tab_ref, w1p_ref, cxt_ref, bw_ref, bx_ref,
              mw_ref, mh_ref, vw_ref, vh_ref):
    first = (pl.program_id(0) == 0) & (pl.program_id(1) == 0)
    d = cxt_ref.shape[0] // 2
    cen = tab_ref[0]
    cend = cen[:, :d]
    mw = mw_ref[...]
    mh = mh_ref[...]
    wq = jnp.zeros((1, HALF), F32)
    hq = jnp.zeros((1, DIM), F32)
    for k in range(K):
        rel = g_ref[0, k] - cen
        w = jnp.dot(rel, w1p_ref[...], preferred_element_type=F32) + bw_ref[...]
        dw = w - mw
        wq += jnp.sum(dw * dw, axis=0, keepdims=True)
        ee = jnp.concatenate([cend, rel[:, :d]], axis=1)
        h = jnp.dot(ee, cxt_ref[...], preferred_element_type=F32) + bx_ref[...]
        dh = h - mh
        hq += jnp.sum(dh * dh, axis=0, keepdims=True)

    @pl.when(first)
    def _():
        vw_ref[...] = jnp.zeros_like(vw_ref)
        vh_ref[...] = jnp.zeros_like(vh_ref)

    vw_ref[...] += wq
    vh_ref[...] += hq


def _e1v(g, tab, w1p, cxt, bw, bx, mw, mh):
    nt = N // NT
    return pl.pallas_call(
        _e1v_body,
        grid=(B, nt),
        in_specs=[
            pl.BlockSpec((1, K, NT, DIM), lambda b, i: (b, 0, i, 0)),
            pl.BlockSpec((1, NT, DIM), lambda b, i: (b, i, 0)),
            pl.BlockSpec((DIM, HALF), lambda b, i: (0, 0)),
            pl.BlockSpec(cxt.shape, lambda b, i: (0, 0)),
            pl.BlockSpec((1, HALF), lambda b, i: (0, 0)),
            pl.BlockSpec((1, DIM), lambda b, i: (0, 0)),
            pl.BlockSpec((1, HALF), lambda b, i: (0, 0)),
            pl.BlockSpec((1, DIM), lambda b, i: (0, 0)),
        ],
        out_specs=[
            pl.BlockSpec((1, HALF), lambda b, i: (0, 0)),
            pl.BlockSpec((1, DIM), lambda b, i: (0, 0)),
        ],
        out_shape=[
            jax.ShapeDtypeStruct((1, HALF), F32),
            jax.ShapeDtypeStruct((1, DIM), F32),
        ],
    )(g, tab, w1p, cxt, bw, bx, mw, mh)


def _e2v_body(w2_ref, m2_ref, v2_ref):
    first = (pl.program_id(0) == 0) & (pl.program_id(1) == 0)
    m2 = m2_ref[...]
    q = jnp.zeros((1, DIM), F32)
    for k in range(K):
        dv = w2_ref[0, k] - m2
        q += jnp.sum(dv * dv, axis=0, keepdims=True)

    @pl.when(first)
    def _():
        v2_ref[...] = jnp.zeros_like(v2_ref)

    v2_ref[...] += q


def _e2v(w2, m2):
    nt = N // NT
    return pl.pallas_call(
        _e2v_body,
        grid=(B, nt),
        in_specs=[
            pl.BlockSpec((1, K, NT, DIM), lambda b, i: (b, 0, i, 0)),
            pl.BlockSpec((1, DIM), lambda b, i: (0, 0)),
        ],
        out_specs=pl.BlockSpec((1, DIM), lambda b, i: (0, 0)),
        out_shape=jax.ShapeDtypeStruct((1, DIM), F32),
    )(w2, m2)


def _e2_body(g_ref, tab_ref, w1p_ref, bw_ref, gw_ref, mw_ref, svw_ref,
             bbw_ref, w2w_ref, w2b_ref, w2_ref, s2_ref):
    first = (pl.program_id(0) == 0) & (pl.program_id(1) == 0)
    cen = tab_ref[0]
    gw = gw_ref[...]
    mw = mw_ref[...]
    svw = svw_ref[...]
    bbw = bbw_ref[...]
    ss = jnp.zeros((1, DIM), F32)
    sq = jnp.zeros((1, DIM), F32)
    for k in range(K):
        rel = g_ref[0, k] - cen
        wpre = jnp.dot(rel, w1p_ref[...], preferred_element_type=F32) + bw_ref[...]
        u = _lrelu(gw * (wpre - mw) / svw + bbw, 0.2)
        w2k = jnp.dot(u, w2w_ref[...], preferred_element_type=F32) + w2b_ref[...]
        w2_ref[0, k] = w2k
        ss += jnp.sum(w2k, axis=0, keepdims=True)
        sq += jnp.sum(w2k * w2k, axis=0, keepdims=True)

    @pl.when(first)
    def _():
        s2_ref[...] = jnp.zeros_like(s2_ref)

    s2_ref[...] += jnp.concatenate([ss, sq], axis=0)


def _e2(g, tab, w1p, bw, gw, mw, svw, bbw, w2w, w2b):
    nt = N // NT
    vec64 = pl.BlockSpec((1, HALF), lambda b, i: (0, 0))
    return pl.pallas_call(
        _e2_body,
        grid=(B, nt),
        in_specs=[
            pl.BlockSpec((1, K, NT, DIM), lambda b, i: (b, 0, i, 0)),
            pl.BlockSpec((1, NT, DIM), lambda b, i: (b, i, 0)),
            pl.BlockSpec((DIM, HALF), lambda b, i: (0, 0)),
            vec64, vec64, vec64, vec64, vec64,
            pl.BlockSpec((HALF, DIM), lambda b, i: (0, 0)),
            pl.BlockSpec((1, DIM), lambda b, i: (0, 0)),
        ],
        out_specs=[
            pl.BlockSpec((1, K, NT, DIM), lambda b, i: (b, 0, i, 0)),
            pl.BlockSpec((2, DIM), lambda b, i: (0, 0)),
        ],
        out_shape=[
            jax.ShapeDtypeStruct((B, K, N, DIM), F32),
            jax.ShapeDtypeStruct((2, DIM), F32),
        ],
    )(g, tab, w1p, bw, gw, mw, svw, bbw, w2w, w2b)


def _e3_body(w2_ref, g_ref, tab_ref, cxt_ref, bx_ref, g2_ref, m2_ref,
             sv2_ref, b2_ref, gh_ref, mh_ref, svh_ref, bh_ref,
             cof_ref, cob_ref, h_ref, sb_ref):
    first = pl.program_id(1) == 0
    d = cxt_ref.shape[0] // 2
    g2 = g2_ref[...]
    m2 = m2_ref[...]
    sv2 = sv2_ref[...]
    b2 = b2_ref[...]
    acts = [_lrelu(g2 * (w2_ref[0, k] - m2) / sv2 + b2, 0.2) for k in range(K)]
    mx = acts[0]
    for k in range(1, K):
        mx = jnp.maximum(mx, acts[k])
    es = [jnp.exp(a - mx) for a in acts]
    tot = es[0]
    for k in range(1, K):
        tot = tot + es[k]
    cen = tab_ref[0]
    cend = cen[:, :d]
    gh = gh_ref[...]
    mh = mh_ref[...]
    svh = svh_ref[...]
    bh = bh_ref[...]
    prods = []
    for k in range(K):
        w = es[k] / tot
        rel = g_ref[0, k] - cen
        ee = jnp.concatenate([cend, rel[:, :d]], axis=1)
        hpre = jnp.dot(ee, cxt_ref[...], preferred_element_type=F32) + bx_ref[...]
        h = _lrelu(gh * (hpre - mh) / svh + bh, 0.2)
        prods.append(h * w)
    stacked = jnp.concatenate(prods, axis=1)
    out = jnp.dot(stacked, cof_ref[...], preferred_element_type=F32) + cob_ref[...]
    h_ref[0] = out

    @pl.when(first)
    def _():
        sb_ref[...] = jnp.zeros_like(sb_ref)

    sb_ref[0] += jnp.sum(out, axis=0, keepdims=True)


def _e3(w2, g, tab, cxt, bx, g2, m2, sv2, b2, gh, mh, svh, bh, cof, cob):
    nt = N // NT
    vec = pl.BlockSpec((1, DIM), lambda b, i: (0, 0))
    return pl.pallas_call(
        _e3_body,
        grid=(B, nt),
        in_specs=[
            pl.BlockSpec((1, K, NT, DIM), lambda b, i: (b, 0, i, 0)),
            pl.BlockSpec((1, K, NT, DIM), lambda b, i: (b, 0, i, 0)),
            pl.BlockSpec((1, NT, DIM), lambda b, i: (b, i, 0)),
            pl.BlockSpec(cxt.shape, lambda b, i: (0, 0)),
            vec, vec, vec, vec, vec, vec, vec, vec, vec,
            pl.BlockSpec((DIM * K, DIM), lambda b, i: (0, 0)),
            vec,
        ],
        out_specs=[
            pl.BlockSpec((1, NT, DIM), lambda b, i: (b, i, 0)),
            pl.BlockSpec((1, 1, DIM), lambda b, i: (b, 0, 0)),
        ],
        out_shape=[
            jax.ShapeDtypeStruct((B, N, DIM), F32),
            jax.ShapeDtypeStruct((B, 1, DIM), F32),
        ],
    )(w2, g, tab, cxt, bx, g2, m2, sv2, b2, gh, mh, svh, bh, cof, cob)


def _e3v_body(h_ref, m_ref, sv_ref):
    first = pl.program_id(1) == 0
    dv = h_ref[0] - m_ref[0]

    @pl.when(first)
    def _():
        sv_ref[...] = jnp.zeros_like(sv_ref)

    sv_ref[0] += jnp.sum(dv * dv, axis=0, keepdims=True)


def _e3v(heb, mb):
    nt = N // NT
    return pl.pallas_call(
        _e3v_body,
        grid=(B, nt),
        in_specs=[
            pl.BlockSpec((1, NT, DIM), lambda b, i: (b, i, 0)),
            pl.BlockSpec((1, 1, DIM), lambda b, i: (b, 0, 0)),
        ],
        out_specs=pl.BlockSpec((1, 1, DIM), lambda b, i: (b, 0, 0)),
        out_shape=jax.ShapeDtypeStruct((B, 1, DIM), F32),
    )(heb, mb)


def _e4_block0_body(h_ref, st_ref, m_ref, sv_ref, adw_ref, adb_ref,
                    h1_ref, h1t_ref):
    xh = (h_ref[0] - m_ref[0]) / sv_ref[0]
    s = jnp.dot(st_ref[0], adw_ref[...], preferred_element_type=F32) + adb_ref[...]
    o = _lrelu(s[:, :DIM] * xh + s[:, DIM:], 0.2)
    h1_ref[0] = o
    h1t_ref[0] = jnp.transpose(o)


def _e4_block0(h, st, m, r, adw, adb):
    nt = N // NT
    return pl.pallas_call(
        _e4_block0_body,
        grid=(B, nt),
        in_specs=[
            pl.BlockSpec((1, NT, DIM), lambda b, i: (b, i, 0)),
            pl.BlockSpec((1, NT, DIM), lambda b, i: (b, i, 0)),
            pl.BlockSpec((1, 1, DIM), lambda b, i: (b, 0, 0)),
            pl.BlockSpec((1, 1, DIM), lambda b, i: (b, 0, 0)),
            pl.BlockSpec((DIM, 2 * DIM), lambda b, i: (0, 0)),
            pl.BlockSpec((1, 2 * DIM), lambda b, i: (0, 0)),
        ],
        out_specs=[
            pl.BlockSpec((1, NT, DIM), lambda b, i: (b, i, 0)),
            pl.BlockSpec((1, DIM, NT), lambda b, i: (b, 0, i)),
        ],
        out_shape=[
            jax.ShapeDtypeStruct((B, N, DIM), F32),
            jax.ShapeDtypeStruct((B, DIM, N), F32),
        ],
    )(h, st, m, r, adw, adb)


def _e4_block1_body(h_ref, st_ref, m_ref, sv_ref, adw_ref, adb_ref,
                    h2_ref, mx_ref):
    i = pl.program_id(1)
    xh = (h_ref[0] - m_ref[0]) / sv_ref[0]
    s = jnp.dot(st_ref[0], adw_ref[...], preferred_element_type=F32) + adb_ref[...]
    o = _lrelu(s[:, :DIM] * xh + s[:, DIM:], 0.2)
    h2_ref[0] = o
    pm = jnp.max(o, axis=0, keepdims=True)

    @pl.when(i == 0)
    def _():
        mx_ref[0] = pm

    @pl.when(i > 0)
    def _():
        mx_ref[0] = jnp.maximum(mx_ref[0], pm)


def _e4_block1(h, st, m, r, adw, adb):
    nt = N // NT
    return pl.pallas_call(
        _e4_block1_body,
        grid=(B, nt),
        in_specs=[
            pl.BlockSpec((1, NT, DIM), lambda b, i: (b, i, 0)),
            pl.BlockSpec((1, NT, DIM), lambda b, i: (b, i, 0)),
            pl.BlockSpec((1, 1, DIM), lambda b, i: (b, 0, 0)),
            pl.BlockSpec((1, 1, DIM), lambda b, i: (b, 0, 0)),
            pl.BlockSpec((DIM, 2 * DIM), lambda b, i: (0, 0)),
            pl.BlockSpec((1, 2 * DIM), lambda b, i: (0, 0)),
        ],
        out_specs=[
            pl.BlockSpec((1, NT, DIM), lambda b, i: (b, i, 0)),
            pl.BlockSpec((1, 1, DIM), lambda b, i: (b, 0, 0)),
        ],
        out_shape=[
            jax.ShapeDtypeStruct((B, N, DIM), F32),
            jax.ShapeDtypeStruct((B, 1, DIM), F32),
        ],
    )(h, st, m, r, adw, adb)


def _g_body(mx_ref, w1_ref, b1_ref, g1_ref, be1_ref, w2_ref, b2_ref,
            g2_ref, be2_ref, fg_ref):
    f = jnp.dot(mx_ref[...], w1_ref[...], preferred_element_type=F32) + b1_ref[...]
    m = jnp.mean(f, axis=0, keepdims=True)
    v = jnp.mean((f - m) * (f - m), axis=0, keepdims=True)
    f = _lrelu(g1_ref[...] * (f - m) / jnp.sqrt(v + EPS) + be1_ref[...], 0.01)
    f = jnp.dot(f, w2_ref[...], preferred_element_type=F32) + b2_ref[...]
    m = jnp.mean(f, axis=0, keepdims=True)
    v = jnp.mean((f - m) * (f - m), axis=0, keepdims=True)
    fg_ref[...] = _lrelu(g2_ref[...] * (f - m) / jnp.sqrt(v + EPS) + be2_ref[...], 0.01)


def _g_head(mx, w1, b1, g1, be1, w2, b2, g2, be2):
    args = (mx, w1, b1, g1, be1, w2, b2, g2, be2)
    return pl.pallas_call(
        _g_body,
        grid=(1,),
        in_specs=[pl.BlockSpec(a.shape, lambda i: (0, 0)) for a in args],
        out_specs=pl.BlockSpec((B, GF), lambda i: (0, 0)),
        out_shape=jax.ShapeDtypeStruct((B, GF), F32),
    )(*args)


def _asm_body(fg_ref, h2_ref, o_ref):
    o_ref[0, :, :GF] = jnp.broadcast_to(fg_ref[0], (NT, GF))
    o_ref[0, :, GF:] = h2_ref[0]


def _assemble(fg, h2):
    fg = fg.reshape(B, 1, GF)
    nt = N // NT
    return pl.pallas_call(
        _asm_body,
        grid=(B, nt),
        in_specs=[
            pl.BlockSpec((1, 1, GF), lambda b, i: (b, 0, 0)),
            pl.BlockSpec((1, NT, DIM), lambda b, i: (b, i, 0)),
        ],
        out_specs=pl.BlockSpec((1, NT, GF + DIM), lambda b, i: (b, i, 0)),
        out_shape=jax.ShapeDtypeStruct((B, N, GF + DIM), F32),
    )(fg, h2)


def _pad_rows(w):
    d = w.shape[0]
    if d == DIM:
        return w
    return jnp.concatenate([w, jnp.zeros((DIM - d, w.shape[1]), F32)], axis=0)


def _edge_block(pfx, h_rows, h_t, d, tab, style, p, adw, adb, extra):
    sq = jnp.sum(h_rows * h_rows, axis=2)
    idx = _knn(h_rows, h_t, sq, d)
    idxf = jnp.transpose(idx, (0, 2, 1)).reshape(R)
    g = _sc_gather(tab.reshape(B * N, DIM), idxf)
    g = g.reshape(B, K, N, DIM)
    cxw = p[pfx + 'cx_w']
    w1p = _pad_rows(p[pfx + 'cw1_w'].T)
    cxt = cxw.T
    bw = p[pfx + 'cw1_b'][None]
    bx = p[pfx + 'cx_b'][None]
    sw, shst = _e1(g, tab, w1p, cxt, bw, bx)
    m_cnt = float(B * K * N)
    mw = sw[0][None] / m_cnt
    mh = shst[0][None] / m_cnt
    vwq, vhq = _e1v(g, tab, w1p, cxt, bw, bx, mw, mh)
    svw = jnp.sqrt(vwq / m_cnt + EPS)
    svh = jnp.sqrt(vhq / m_cnt + EPS)
    w2, s2 = _e2(g, tab, w1p, bw, p[pfx + 'bnw1_g'][None], mw, svw,
                 p[pfx + 'bnw1_b'][None], p[pfx + 'cw2_w'].T,
                 p[pfx + 'cw2_b'][None])
    m2 = s2[0][None] / m_cnt
    v2q = _e2v(w2, m2)
    sv2 = jnp.sqrt(v2q / m_cnt + EPS)
    cof = jnp.transpose(p[pfx + 'co_w'], (2, 1, 0)).reshape(K * DIM, DIM)
    heb, sb = _e3(w2, g, tab, cxt, bx,
                  p[pfx + 'bnw2_g'][None], m2, sv2, p[pfx + 'bnw2_b'][None],
                  p[pfx + 'bnx_g'][None], mh, svh, p[pfx + 'bnx_b'][None],
                  cof, p[pfx + 'co_b'][None])
    mb = sb / N
    vq = _e3v(heb, mb)
    svb = jnp.sqrt(vq / N + EPS)
    if extra:
        return _e4_block0(heb, style, mb, svb, adw, adb)
    return _e4_block1(heb, style, mb, svb, adw, adb)


def kernel(x, z, params):
    p = params
    style, tab0 = _style_tab0(
        x, z, p['head_w1'].T, p['head_b1'][None], p['head_w2'].T,
        p['head_b2'][None])

    h1, h1t = _edge_block(
        'eb0_', x, jnp.transpose(x, (0, 2, 1)), 3, tab0, style, p,
        p['ad0_w'].T, p['ad0_b'][None], extra=True)

    h2, mx = _edge_block(
        'eb1_', h1, h1t, DIM, h1, style, p,
        p['ad1_w'].T, p['ad1_b'][None], extra=False)

    fg = _g_head(mx.reshape(B, DIM),
                 p['g_w1'].T, p['g_b1'][None], p['g_bn1_g'][None],
                 p['g_bn1_b'][None],
                 p['g_w2'].T, p['g_b2'][None], p['g_bn2_g'][None],
                 p['g_bn2_b'][None])
    return _assemble(fg, h2)
